# baseline (speedup 1.0000x reference)
import functools

import jax
import jax.numpy as jnp
from jax import lax
from jax.experimental import pallas as pl
from jax.experimental.pallas import tpu as pltpu
from jax.experimental.pallas import tpu_sc as plsc

_NC = 2
_NS = 16
_NW = _NC * _NS
_L = 16

_CS = 16


def _bcast_lane0(vec):
  dims = lax.GatherDimensionNumbers(
      offset_dims=(), collapsed_slice_dims=(0,), start_index_map=(0,))
  idx = jnp.zeros((_L, 1), jnp.int32)
  return lax.gather(vec, idx, dims, (1,),
                    mode=lax.GatherScatterMode.PROMISE_IN_BOUNDS)


def _make_sc_scatter(n_rows, per_w, n_types, d):
  n_ir = per_w // 128
  cap = per_w // _CS + n_types
  d_vec = d // _L
  mesh = plsc.VectorSubcoreMesh(core_axis_name="c", subcore_axis_name="s")

  @functools.partial(
      pl.kernel,
      out_type=jax.ShapeDtypeStruct((n_rows, d), jnp.float32),
      mesh=mesh,
      compiler_params=pltpu.CompilerParams(needs_layout_passes=False),
      scratch_types=[
          pltpu.VMEM((n_ir, 128), jnp.int32),
          pltpu.VMEM((cap, _CS), jnp.int32),
          pltpu.VMEM((n_types, d), jnp.float32),
          pltpu.VMEM((2, _CS, d), jnp.float32),
          pltpu.SemaphoreType.DMA,
          pltpu.SemaphoreType.DMA,
      ],
  )
  def k(ids_hbm, table_hbm, out_hbm, idx_v, pos_v, table_v, bufs, osem, isem):
    wid = lax.axis_index("s") * _NC + lax.axis_index("c")
    idx_cp = pltpu.async_copy(ids_hbm.at[wid], idx_v, isem)
    pltpu.sync_copy(table_hbm, table_v)
    lanes = lax.iota(jnp.int32, _L)
    wbase = wid * per_w

    def prefill(slot, t):
      row = [table_v[t, pl.ds(c * _L, _L)] for c in range(d_vec)]

      def fill_row(r, c):
        for cc in range(d_vec):
          bufs[slot, r, pl.ds(cc * _L, _L)] = row[cc]
        return c

      lax.fori_loop(0, _CS, fill_row, 0)

    prefill(0, 0)
    prefill(1, 1)
    idx_cp.wait()

    def count_body(r, accs):
      accs = list(accs)
      for q in range(128 // _L):
        v = idx_v[r, pl.ds(q * _L, _L)]
        for t in range(n_types):
          accs[t] = accs[t] + (v == t).astype(jnp.int32)
      return tuple(accs)

    zero_v = jnp.zeros((_L,), jnp.int32)
    accs = lax.fori_loop(0, n_ir, count_body,
                         tuple(zero_v for _ in range(n_types)))
    cnts = [jnp.cumsum(accs[t])[_L - 1] for t in range(n_types)]

    m = [(cnts[t] + (_CS - 1)) // _CS for t in range(n_types)]
    base_c = []
    acc = jnp.int32(0)
    for t in range(n_types):
      base_c.append(acc)
      acc = acc + m[t]

    def fill_body(r, ptrs):
      ptrs = list(ptrs)
      for q in range(128 // _L):
        v = idx_v[r, pl.ds(q * _L, _L)]
        pos = wbase + r * 128 + q * _L + lanes
        for t in range(n_types):
          msk = v == t
          ranks = jnp.cumsum(msk.astype(jnp.int32))
          dest = base_c[t] * _CS + ptrs[t] + ranks - 1
          plsc.store_scatter(pos_v, [dest // _CS, dest % _CS], pos, mask=msk)
          ptrs[t] = ptrs[t] + ranks[_L - 1]
      return tuple(ptrs)

    lax.fori_loop(0, n_ir, fill_body,
                  tuple(jnp.int32(0) for _ in range(n_types)))

    for t in range(n_types):
      rem = cnts[t] % _CS

      @pl.when(rem != 0)
      def _(t=t, rem=rem):
        first = _bcast_lane0(pos_v[base_c[t], pl.ds(0, _L)])
        last_row = base_c[t] + cnts[t] // _CS
        for q in range(_CS // _L):
          lane_ids = q * _L + lanes
          plsc.store_scatter(
              pos_v, [jnp.full((_L,), last_row, jnp.int32), lane_ids],
              first, mask=lane_ids >= rem)

    def drain(n_chunks):
      def w(_, c):
        pltpu.make_async_copy(
            bufs.at[0], out_hbm.at[pos_v.at[0]], osem).wait()
        return c
      lax.fori_loop(0, n_chunks, w, 0)

    for t in range(n_types):
      slot = t % 2
      if t >= 2:
        drain(m[t - 2])
        prefill(slot, t)

      def fire(j, c, t=t, slot=slot):
        pltpu.async_copy(
            bufs.at[slot], out_hbm.at[pos_v.at[base_c[t] + j]], osem)
        return c

      lax.fori_loop(0, m[t], fire, 0)

    drain(m[n_types - 2] + m[n_types - 1])

  return k


def kernel(edge_type_ids, embedding):
  orig_shape = edge_type_ids.shape
  n_types, d = embedding.shape
  flat = edge_type_ids.reshape(-1)
  n = flat.shape[0]

  per = _NW * 128
  n_pad = (-n) % per
  if n_pad:
    flat = jnp.concatenate(
        [flat, jnp.full((n_pad,), n_types, jnp.int32)])
  total = n + n_pad
  per_w = total // _NW
  ids3d = flat.reshape(_NW, per_w // 128, 128)

  out = _make_sc_scatter(n, per_w, n_types, d)(ids3d, embedding)
  return out.reshape(*orig_shape, d)


# speedup vs baseline: 1.0277x; 1.0277x over previous
import functools

import jax
import jax.numpy as jnp
from jax import lax
from jax.experimental import pallas as pl
from jax.experimental.pallas import tpu as pltpu
from jax.experimental.pallas import tpu_sc as plsc

_NC = 2
_NS = 16
_NW = _NC * _NS
_L = 16

_CS = 16


def _bcast_lane0(vec):
  dims = lax.GatherDimensionNumbers(
      offset_dims=(), collapsed_slice_dims=(0,), start_index_map=(0,))
  idx = jnp.zeros((_L, 1), jnp.int32)
  return lax.gather(vec, idx, dims, (1,),
                    mode=lax.GatherScatterMode.PROMISE_IN_BOUNDS)


def _make_sc_scatter(n_rows, per_w, n_types, d):
  n_ir = per_w // 128
  cap = per_w // _CS + n_types
  d_vec = d // _L
  mesh = plsc.VectorSubcoreMesh(core_axis_name="c", subcore_axis_name="s")

  @functools.partial(
      pl.kernel,
      out_type=jax.ShapeDtypeStruct((n_rows, d), jnp.float32),
      mesh=mesh,
      compiler_params=pltpu.CompilerParams(needs_layout_passes=False),
      scratch_types=[
          pltpu.VMEM((n_ir, 128), jnp.int32),
          pltpu.VMEM((cap, _CS), jnp.int32),
          pltpu.VMEM((n_types, d), jnp.float32),
          pltpu.VMEM((2, _CS, d), jnp.float32),
          pltpu.SemaphoreType.DMA,
          pltpu.SemaphoreType.DMA,
      ],
  )
  def k(ids_hbm, table_hbm, out_hbm, idx_v, pos_v, table_v, bufs, osem, isem):
    wid = lax.axis_index("s") * _NC + lax.axis_index("c")
    idx_cp = pltpu.async_copy(ids_hbm.at[wid], idx_v, isem)
    pltpu.sync_copy(table_hbm, table_v)
    lanes = lax.iota(jnp.int32, _L)
    wbase = wid * per_w

    def prefill(slot, t):
      row = [table_v[t, pl.ds(c * _L, _L)] for c in range(d_vec)]

      def fill_row(r, c):
        for cc in range(d_vec):
          bufs[slot, r, pl.ds(cc * _L, _L)] = row[cc]
        return c

      lax.fori_loop(0, _CS, fill_row, 0)

    prefill(0, 0)
    prefill(1, 1)
    idx_cp.wait()

    def count_body(r, cnts):
      cnts = list(cnts)
      for q in range(128 // _L):
        v = idx_v[r, pl.ds(q * _L, _L)]
        for t in range(n_types):
          cs = jnp.cumsum((v == t).astype(jnp.int32))
          cnts[t] = cnts[t] + cs[_L - 1]
      return tuple(cnts)

    cnts = lax.fori_loop(0, n_ir, count_body,
                         tuple(jnp.int32(0) for _ in range(n_types)))

    m = [(cnts[t] + (_CS - 1)) // _CS for t in range(n_types)]
    base_c = []
    acc = jnp.int32(0)
    for t in range(n_types):
      base_c.append(acc)
      acc = acc + m[t]

    def fill_body(r, ptrs):
      ptrs = list(ptrs)
      for q in range(128 // _L):
        v = idx_v[r, pl.ds(q * _L, _L)]
        pos = wbase + r * 128 + q * _L + lanes
        for t in range(n_types):
          msk = v == t
          ranks = jnp.cumsum(msk.astype(jnp.int32))
          dest = base_c[t] * _CS + ptrs[t] + ranks - 1
          plsc.store_scatter(pos_v, [dest // _CS, dest % _CS], pos, mask=msk)
          ptrs[t] = ptrs[t] + ranks[_L - 1]
      return tuple(ptrs)

    lax.fori_loop(0, n_ir, fill_body,
                  tuple(jnp.int32(0) for _ in range(n_types)))

    for t in range(n_types):
      rem = cnts[t] % _CS

      @pl.when(rem != 0)
      def _(t=t, rem=rem):
        first = _bcast_lane0(pos_v[base_c[t], pl.ds(0, _L)])
        last_row = base_c[t] + cnts[t] // _CS
        for q in range(_CS // _L):
          lane_ids = q * _L + lanes
          plsc.store_scatter(
              pos_v, [jnp.full((_L,), last_row, jnp.int32), lane_ids],
              first, mask=lane_ids >= rem)

    def drain(n_chunks):
      def w(_, c):
        pltpu.make_async_copy(
            bufs.at[0], out_hbm.at[pos_v.at[0]], osem).wait()
        return c
      lax.fori_loop(0, n_chunks, w, 0)

    for t in range(n_types):
      slot = t % 2
      if t >= 2:
        drain(m[t - 2])
        prefill(slot, t)

      def fire(j, c, t=t, slot=slot):
        pltpu.async_copy(
            bufs.at[slot], out_hbm.at[pos_v.at[base_c[t] + j]], osem)
        return c

      lax.fori_loop(0, m[t], fire, 0)

    drain(m[n_types - 2] + m[n_types - 1])

  return k


def kernel(edge_type_ids, embedding):
  orig_shape = edge_type_ids.shape
  n_types, d = embedding.shape
  flat = edge_type_ids.reshape(-1)
  n = flat.shape[0]

  per = _NW * 128
  n_pad = (-n) % per
  if n_pad:
    flat = jnp.concatenate(
        [flat, jnp.full((n_pad,), n_types, jnp.int32)])
  total = n + n_pad
  per_w = total // _NW
  ids3d = flat.reshape(_NW, per_w // 128, 128)

  out = _make_sc_scatter(n, per_w, n_types, d)(ids3d, embedding)
  return out.reshape(*orig_shape, d)
